# baseline (speedup 1.0000x reference)
import functools

import jax
import jax.numpy as jnp
from jax import lax
from jax.experimental import pallas as pl
from jax.experimental.pallas import tpu as pltpu
from jax.experimental.pallas import tpu_sc as plsc

_LANES = 16
_NUM_WORKERS = 32
_F = 4


def _pick_chunk_groups(base_groups: int) -> int:
    for d in range(min(512, base_groups), 0, -1):
        if base_groups % d == 0:
            return d
    return 1


def _compute_groups(x_ref, out_ref, params, n_groups):
    lo = params[0:4]
    hi = params[4:8]
    flip = tuple(params[8 + f] != 0.0 for f in range(4))
    iota = lax.iota(jnp.int32, _LANES)
    col = tuple(iota * _F + f for f in range(4))

    def body(i, idx0):
        p = None
        for f in range(4):
            a = plsc.load_gather(x_ref, [idx0 + col[f]])
            pf = jnp.logical_xor((a >= lo[f]) & (a <= hi[f]), flip[f])
            p = pf if p is None else (p & pf)
        out_ref[pl.ds(i * _LANES, _LANES)] = jnp.where(p, 1.0, 0.0).astype(
            jnp.float32)
        return idx0 + _LANES * _F

    lax.fori_loop(0, n_groups, body, jnp.int32(0), unroll=4)


def _make_sc_kernel(n_rows: int):
    total_g = n_rows // _LANES
    base_g = total_g // _NUM_WORKERS
    rem_g = total_g % _NUM_WORKERS
    chunk_g = _pick_chunk_groups(base_g)
    n_chunks = base_g // chunk_g
    chunk_words = chunk_g * _LANES * _F
    chunk_out = chunk_g * _LANES

    mesh = plsc.VectorSubcoreMesh(core_axis_name="c", subcore_axis_name="s")

    @functools.partial(
        pl.kernel,
        mesh=mesh,
        out_type=jax.ShapeDtypeStruct((n_rows,), jnp.float32),
        compiler_params=pltpu.CompilerParams(needs_layout_passes=False),
        scratch_types=[
            pltpu.VMEM((chunk_words,), jnp.float32),
            pltpu.VMEM((chunk_words,), jnp.float32),
            pltpu.VMEM((chunk_out,), jnp.float32),
            pltpu.VMEM((chunk_out,), jnp.float32),
            pltpu.VMEM((12, _LANES), jnp.float32),
            pltpu.SemaphoreType.DMA,
            pltpu.SemaphoreType.DMA,
            pltpu.SemaphoreType.DMA,
            pltpu.SemaphoreType.DMA,
        ],
    )
    def sc_kernel(x_hbm, par_hbm, out_hbm, xb0, xb1, yb0, yb1, pbuf, si0, si1,
                  so0, so1):
        wid = lax.axis_index("s") * 2 + lax.axis_index("c")
        my_g0 = wid * base_g + jnp.minimum(wid, rem_g)

        pltpu.sync_copy(par_hbm, pbuf)
        params = tuple(pbuf[i, :] for i in range(12))
        xbufs = (xb0, xb1)
        ybufs = (yb0, yb1)
        in_sems = (si0, si1)
        out_sems = (so0, so1)

        def in_copy(k, slot):
            g = (my_g0 + k * chunk_g) * _LANES * _F
            return pltpu.async_copy(
                x_hbm.at[pl.ds(g, chunk_words)], xbufs[slot], in_sems[slot])

        def out_copy(k, slot):
            g = (my_g0 + k * chunk_g) * _LANES
            return pltpu.async_copy(
                ybufs[slot], out_hbm.at[pl.ds(g, chunk_out)], out_sems[slot])

        in_flight = {0: in_copy(0, 0)}
        out_flight = {}
        for k in range(n_chunks):
            slot = k % 2
            if k + 1 < n_chunks:
                in_flight[k + 1] = in_copy(k + 1, 1 - slot)
            in_flight.pop(k).wait()
            if k >= 2:
                out_flight.pop(k - 2).wait()
            _compute_groups(xbufs[slot], ybufs[slot], params, chunk_g)
            out_flight[k] = out_copy(k, slot)
        for k in sorted(out_flight):
            out_flight.pop(k).wait()

        @pl.when(wid < rem_g)
        def _():
            g_extra = my_g0 + base_g
            pltpu.sync_copy(
                x_hbm.at[pl.ds(g_extra * _LANES * _F, _LANES * _F)],
                xb0.at[pl.ds(0, _LANES * _F)])
            _compute_groups(xb0, yb0, params, 1)
            pltpu.sync_copy(
                yb0.at[pl.ds(0, _LANES)],
                out_hbm.at[pl.ds(g_extra * _LANES, _LANES)])

    return sc_kernel


def kernel(x, cuts, cases):
    n, f = x.shape
    assert f == _F and n % _LANES == 0
    c0 = cuts[:, 0]
    c1 = cuts[:, 1]
    inf = jnp.float32(jnp.inf)
    lo = jnp.where(cases == 0, -inf,
                   jnp.where(cases == 3, jnp.nextafter(c0, inf), c0))
    hi = jnp.where(cases == 0, c0,
                   jnp.where(cases == 1, inf,
                             jnp.where(cases == 2, c1,
                                       jnp.nextafter(c1, -inf))))
    flip = (cases == 3).astype(jnp.float32)
    params = jnp.concatenate(
        [jnp.broadcast_to(lo[:, None], (4, _LANES)),
         jnp.broadcast_to(hi[:, None], (4, _LANES)),
         jnp.broadcast_to(flip[:, None], (4, _LANES))], axis=0)

    sc = _make_sc_kernel(n)
    return sc(x.reshape(-1), params)


# speedup vs baseline: 42.5686x; 42.5686x over previous
import functools

import jax
import jax.numpy as jnp
from jax import lax
from jax.experimental import pallas as pl
from jax.experimental.pallas import tpu as pltpu
from jax.experimental.pallas import tpu_sc as plsc

_LANES = 16
_NUM_WORKERS = 32
_F = 4
_TILE = 128


def _pick_chunk_tiles(base_tiles: int) -> int:
    for d in range(min(64, base_tiles), 0, -1):
        if base_tiles % d == 0:
            return d
    return 1


def _compute_tiles(x_ref, out_ref, params, n_tiles):
    lo = params[0:4]
    hi = params[4:8]
    flip = tuple(params[8 + f] != 0.0 for f in range(4))

    def body(t, _):
        for j in range(_TILE // _LANES):
            p = None
            for f in range(4):
                a = x_ref[t, f, pl.ds(j * _LANES, _LANES)]
                pf = jnp.logical_xor((a >= lo[f]) & (a <= hi[f]), flip[f])
                p = pf if p is None else (p & pf)
            out_ref[pl.ds(t * _TILE + j * _LANES, _LANES)] = jnp.where(
                p, 1.0, 0.0).astype(jnp.float32)
        return 0

    lax.fori_loop(0, n_tiles, body, 0)


def _make_sc_kernel(n_rows: int):
    total_t = n_rows // _TILE
    base_t = total_t // _NUM_WORKERS
    rem_t = total_t % _NUM_WORKERS
    chunk_t = _pick_chunk_tiles(base_t)
    n_chunks = base_t // chunk_t
    chunk_out = chunk_t * _TILE

    mesh = plsc.VectorSubcoreMesh(core_axis_name="c", subcore_axis_name="s")

    @functools.partial(
        pl.kernel,
        mesh=mesh,
        out_type=jax.ShapeDtypeStruct((n_rows,), jnp.float32),
        compiler_params=pltpu.CompilerParams(needs_layout_passes=False),
        scratch_types=[
            pltpu.VMEM((chunk_t, _F, _TILE), jnp.float32),
            pltpu.VMEM((chunk_t, _F, _TILE), jnp.float32),
            pltpu.VMEM((chunk_out,), jnp.float32),
            pltpu.VMEM((chunk_out,), jnp.float32),
            pltpu.VMEM((12, _LANES), jnp.float32),
            pltpu.SemaphoreType.DMA,
            pltpu.SemaphoreType.DMA,
            pltpu.SemaphoreType.DMA,
            pltpu.SemaphoreType.DMA,
        ],
    )
    def sc_kernel(x_hbm, par_hbm, out_hbm, xb0, xb1, yb0, yb1, pbuf, si0, si1,
                  so0, so1):
        wid = lax.axis_index("s") * 2 + lax.axis_index("c")
        my_t0 = wid * base_t + jnp.minimum(wid, rem_t)

        pltpu.sync_copy(par_hbm, pbuf)
        params = tuple(pbuf[i, :] for i in range(12))
        xbufs = (xb0, xb1)
        ybufs = (yb0, yb1)
        in_sems = (si0, si1)
        out_sems = (so0, so1)

        def in_copy(k, slot):
            t = my_t0 + k * chunk_t
            return pltpu.async_copy(
                x_hbm.at[pl.ds(t, chunk_t), :, :], xbufs[slot], in_sems[slot])

        def out_copy(k, slot):
            w = (my_t0 + k * chunk_t) * _TILE
            return pltpu.async_copy(
                ybufs[slot], out_hbm.at[pl.ds(w, chunk_out)], out_sems[slot])

        in_flight = {0: in_copy(0, 0)}
        out_flight = {}
        for k in range(n_chunks):
            slot = k % 2
            if k + 1 < n_chunks:
                in_flight[k + 1] = in_copy(k + 1, 1 - slot)
            in_flight.pop(k).wait()
            if k >= 2:
                out_flight.pop(k - 2).wait()
            _compute_tiles(xbufs[slot], ybufs[slot], params, chunk_t)
            out_flight[k] = out_copy(k, slot)
        for k in sorted(out_flight):
            out_flight.pop(k).wait()

        @pl.when(wid < rem_t)
        def _():
            t_extra = my_t0 + base_t
            pltpu.sync_copy(x_hbm.at[pl.ds(t_extra, 1), :, :],
                            xb0.at[pl.ds(0, 1), :, :])
            _compute_tiles(xb0, yb0, params, 1)
            pltpu.sync_copy(yb0.at[pl.ds(0, _TILE)],
                            out_hbm.at[pl.ds(t_extra * _TILE, _TILE)])

    return sc_kernel


def kernel(x, cuts, cases):
    n, f = x.shape
    assert f == _F and n % _TILE == 0
    c0 = cuts[:, 0]
    c1 = cuts[:, 1]
    inf = jnp.float32(jnp.inf)
    lo = jnp.where(cases == 0, -inf,
                   jnp.where(cases == 3, jnp.nextafter(c0, inf), c0))
    hi = jnp.where(cases == 0, c0,
                   jnp.where(cases == 1, inf,
                             jnp.where(cases == 2, c1,
                                       jnp.nextafter(c1, -inf))))
    flip = (cases == 3).astype(jnp.float32)
    params = jnp.concatenate(
        [jnp.broadcast_to(lo[:, None], (4, _LANES)),
         jnp.broadcast_to(hi[:, None], (4, _LANES)),
         jnp.broadcast_to(flip[:, None], (4, _LANES))], axis=0)

    x_tiles = jnp.swapaxes(x.reshape(n // _TILE, _TILE, _F), 1, 2)

    sc = _make_sc_kernel(n)
    return sc(x_tiles, params)


# speedup vs baseline: 54.3196x; 1.2760x over previous
import functools

import jax
import jax.numpy as jnp
from jax import lax
from jax.experimental import pallas as pl
from jax.experimental.pallas import tpu as pltpu
from jax.experimental.pallas import tpu_sc as plsc

_LANES = 16
_NUM_WORKERS = 32
_F = 4
_TILE = 128


def _pick_chunk_tiles(base_tiles: int) -> int:
    for d in range(min(64, base_tiles), 0, -1):
        if base_tiles % d == 0:
            return d
    return 1


def _compute_tiles(x_ref, out_ref, params, n_tiles):
    lo = params[0:4]
    hi = params[4:8]
    flip = tuple(params[8 + f] != 0.0 for f in range(4))

    @plsc.parallel_loop(0, n_tiles, unroll=2)
    def _(t):
        for j in range(_TILE // _LANES):
            p = None
            for f in range(4):
                a = x_ref[t, f, pl.ds(j * _LANES, _LANES)]
                pf = jnp.logical_xor((a >= lo[f]) & (a <= hi[f]), flip[f])
                p = pf if p is None else (p & pf)
            out_ref[pl.ds(t * _TILE + j * _LANES, _LANES)] = jnp.where(
                p, 1.0, 0.0).astype(jnp.float32)


def _make_sc_kernel(n_rows: int):
    total_t = n_rows // _TILE
    base_t = total_t // _NUM_WORKERS
    rem_t = total_t % _NUM_WORKERS
    chunk_t = _pick_chunk_tiles(base_t)
    n_chunks = base_t // chunk_t
    chunk_out = chunk_t * _TILE

    mesh = plsc.VectorSubcoreMesh(core_axis_name="c", subcore_axis_name="s")

    @functools.partial(
        pl.kernel,
        mesh=mesh,
        out_type=jax.ShapeDtypeStruct((n_rows,), jnp.float32),
        compiler_params=pltpu.CompilerParams(needs_layout_passes=False),
        scratch_types=[
            pltpu.VMEM((chunk_t, _F, _TILE), jnp.float32),
            pltpu.VMEM((chunk_t, _F, _TILE), jnp.float32),
            pltpu.VMEM((chunk_out,), jnp.float32),
            pltpu.VMEM((chunk_out,), jnp.float32),
            pltpu.VMEM((12, _LANES), jnp.float32),
            pltpu.SemaphoreType.DMA,
            pltpu.SemaphoreType.DMA,
            pltpu.SemaphoreType.DMA,
            pltpu.SemaphoreType.DMA,
        ],
    )
    def sc_kernel(x_hbm, par_hbm, out_hbm, xb0, xb1, yb0, yb1, pbuf, si0, si1,
                  so0, so1):
        wid = lax.axis_index("s") * 2 + lax.axis_index("c")
        my_t0 = wid * base_t + jnp.minimum(wid, rem_t)

        pltpu.sync_copy(par_hbm, pbuf)
        params = tuple(pbuf[i, :] for i in range(12))
        xbufs = (xb0, xb1)
        ybufs = (yb0, yb1)
        in_sems = (si0, si1)
        out_sems = (so0, so1)

        def in_copy(k, slot):
            t = my_t0 + k * chunk_t
            return pltpu.async_copy(
                x_hbm.at[pl.ds(t, chunk_t), :, :], xbufs[slot], in_sems[slot])

        def out_copy(k, slot):
            w = (my_t0 + k * chunk_t) * _TILE
            return pltpu.async_copy(
                ybufs[slot], out_hbm.at[pl.ds(w, chunk_out)], out_sems[slot])

        in_flight = {0: in_copy(0, 0)}
        out_flight = {}
        for k in range(n_chunks):
            slot = k % 2
            if k + 1 < n_chunks:
                in_flight[k + 1] = in_copy(k + 1, 1 - slot)
            in_flight.pop(k).wait()
            if k >= 2:
                out_flight.pop(k - 2).wait()
            _compute_tiles(xbufs[slot], ybufs[slot], params, chunk_t)
            out_flight[k] = out_copy(k, slot)
        for k in sorted(out_flight):
            out_flight.pop(k).wait()

        @pl.when(wid < rem_t)
        def _():
            t_extra = my_t0 + base_t
            pltpu.sync_copy(x_hbm.at[pl.ds(t_extra, 1), :, :],
                            xb0.at[pl.ds(0, 1), :, :])
            _compute_tiles(xb0, yb0, params, 1)
            pltpu.sync_copy(yb0.at[pl.ds(0, _TILE)],
                            out_hbm.at[pl.ds(t_extra * _TILE, _TILE)])

    return sc_kernel


def kernel(x, cuts, cases):
    n, f = x.shape
    assert f == _F and n % _TILE == 0
    c0 = cuts[:, 0]
    c1 = cuts[:, 1]
    inf = jnp.float32(jnp.inf)
    lo = jnp.where(cases == 0, -inf,
                   jnp.where(cases == 3, jnp.nextafter(c0, inf), c0))
    hi = jnp.where(cases == 0, c0,
                   jnp.where(cases == 1, inf,
                             jnp.where(cases == 2, c1,
                                       jnp.nextafter(c1, -inf))))
    flip = (cases == 3).astype(jnp.float32)
    params = jnp.concatenate(
        [jnp.broadcast_to(lo[:, None], (4, _LANES)),
         jnp.broadcast_to(hi[:, None], (4, _LANES)),
         jnp.broadcast_to(flip[:, None], (4, _LANES))], axis=0)

    x_tiles = jnp.swapaxes(x.reshape(n // _TILE, _TILE, _F), 1, 2)

    sc = _make_sc_kernel(n)
    return sc(x_tiles, params)


# speedup vs baseline: 59.5927x; 1.0971x over previous
import functools

import jax
import jax.numpy as jnp
from jax import lax
from jax.experimental import pallas as pl
from jax.experimental.pallas import tpu as pltpu
from jax.experimental.pallas import tpu_sc as plsc

_LANES = 16
_NUM_WORKERS = 32
_F = 4
_TILE = 128


def _pick_chunk_tiles(base_tiles: int) -> int:
    for d in range(min(64, base_tiles), 0, -1):
        if base_tiles % d == 0:
            return d
    return 1


def _compute_tiles(x_ref, out_ref, params, n_tiles):
    lo = params[0:4]
    hi = params[4:8]
    flip = tuple(params[8 + f] != 0.0 for f in range(4))

    groups_per_tile = _TILE // _LANES

    @plsc.parallel_loop(0, n_tiles * groups_per_tile, unroll=4)
    def _(v):
        t = v // groups_per_tile
        j = v % groups_per_tile
        p = None
        for f in range(4):
            a = x_ref[t, f, pl.ds(j * _LANES, _LANES)]
            pf = jnp.logical_xor((a >= lo[f]) & (a <= hi[f]), flip[f])
            p = pf if p is None else (p & pf)
        out_ref[pl.ds(v * _LANES, _LANES)] = jnp.where(
            p, 1.0, 0.0).astype(jnp.float32)


def _make_sc_kernel(n_rows: int):
    total_t = n_rows // _TILE
    base_t = total_t // _NUM_WORKERS
    rem_t = total_t % _NUM_WORKERS
    chunk_t = _pick_chunk_tiles(base_t)
    n_chunks = base_t // chunk_t
    chunk_out = chunk_t * _TILE

    mesh = plsc.VectorSubcoreMesh(core_axis_name="c", subcore_axis_name="s")

    @functools.partial(
        pl.kernel,
        mesh=mesh,
        out_type=jax.ShapeDtypeStruct((n_rows,), jnp.float32),
        compiler_params=pltpu.CompilerParams(needs_layout_passes=False),
        scratch_types=[
            pltpu.VMEM((chunk_t, _F, _TILE), jnp.float32),
            pltpu.VMEM((chunk_t, _F, _TILE), jnp.float32),
            pltpu.VMEM((chunk_out,), jnp.float32),
            pltpu.VMEM((chunk_out,), jnp.float32),
            pltpu.VMEM((12, _LANES), jnp.float32),
            pltpu.SemaphoreType.DMA,
            pltpu.SemaphoreType.DMA,
            pltpu.SemaphoreType.DMA,
            pltpu.SemaphoreType.DMA,
        ],
    )
    def sc_kernel(x_hbm, par_hbm, out_hbm, xb0, xb1, yb0, yb1, pbuf, si0, si1,
                  so0, so1):
        wid = lax.axis_index("s") * 2 + lax.axis_index("c")
        my_t0 = wid * base_t + jnp.minimum(wid, rem_t)

        pltpu.sync_copy(par_hbm, pbuf)
        params = tuple(pbuf[i, :] for i in range(12))
        xbufs = (xb0, xb1)
        ybufs = (yb0, yb1)
        in_sems = (si0, si1)
        out_sems = (so0, so1)

        def in_copy(k, slot):
            t = my_t0 + k * chunk_t
            return pltpu.async_copy(
                x_hbm.at[pl.ds(t, chunk_t), :, :], xbufs[slot], in_sems[slot])

        def out_copy(k, slot):
            w = (my_t0 + k * chunk_t) * _TILE
            return pltpu.async_copy(
                ybufs[slot], out_hbm.at[pl.ds(w, chunk_out)], out_sems[slot])

        in_flight = {0: in_copy(0, 0)}
        out_flight = {}
        for k in range(n_chunks):
            slot = k % 2
            if k + 1 < n_chunks:
                in_flight[k + 1] = in_copy(k + 1, 1 - slot)
            in_flight.pop(k).wait()
            if k >= 2:
                out_flight.pop(k - 2).wait()
            _compute_tiles(xbufs[slot], ybufs[slot], params, chunk_t)
            out_flight[k] = out_copy(k, slot)
        for k in sorted(out_flight):
            out_flight.pop(k).wait()

        @pl.when(wid < rem_t)
        def _():
            t_extra = my_t0 + base_t
            pltpu.sync_copy(x_hbm.at[pl.ds(t_extra, 1), :, :],
                            xb0.at[pl.ds(0, 1), :, :])
            _compute_tiles(xb0, yb0, params, 1)
            pltpu.sync_copy(yb0.at[pl.ds(0, _TILE)],
                            out_hbm.at[pl.ds(t_extra * _TILE, _TILE)])

    return sc_kernel


def kernel(x, cuts, cases):
    n, f = x.shape
    assert f == _F and n % _TILE == 0
    c0 = cuts[:, 0]
    c1 = cuts[:, 1]
    inf = jnp.float32(jnp.inf)
    lo = jnp.where(cases == 0, -inf,
                   jnp.where(cases == 3, jnp.nextafter(c0, inf), c0))
    hi = jnp.where(cases == 0, c0,
                   jnp.where(cases == 1, inf,
                             jnp.where(cases == 2, c1,
                                       jnp.nextafter(c1, -inf))))
    flip = (cases == 3).astype(jnp.float32)
    params = jnp.concatenate(
        [jnp.broadcast_to(lo[:, None], (4, _LANES)),
         jnp.broadcast_to(hi[:, None], (4, _LANES)),
         jnp.broadcast_to(flip[:, None], (4, _LANES))], axis=0)

    x_tiles = jnp.swapaxes(x.reshape(n // _TILE, _TILE, _F), 1, 2)

    sc = _make_sc_kernel(n)
    return sc(x_tiles, params)
